# baseline (speedup 1.0000x reference)
import jax
import jax.numpy as jnp
from jax.experimental import pallas as pl


_CHUNK = 512


def _chamfer_kernel(a_ref, bm_ref, out_x_ref, out_y_ref):
    n = bm_ref.shape[1]
    n_chunks = n // _CHUNK

    ymin = jnp.full((n,), jnp.inf, dtype=jnp.float32)
    for i in range(n_chunks):
        a_chunk = a_ref[pl.ds(i * _CHUNK, _CHUNK), :]
        t = jax.lax.dot_general(
            a_chunk,
            bm_ref[...],
            ((( 1,), (0,)), ((), ())),
            preferred_element_type=jnp.float32,
            precision=jax.lax.Precision.HIGHEST,
        )
        out_x_ref[0, pl.ds(i * _CHUNK, _CHUNK)] = jnp.min(t, axis=1)
        ymin = jnp.minimum(ymin, jnp.min(t, axis=0))
    out_y_ref[0, :] = ymin


def kernel(x, y):
    b, d, n = x.shape
    k = d + 2

    xt = jnp.transpose(x, (0, 2, 1))
    nx = jnp.sum(xt * xt, axis=2, keepdims=True)
    ones_x = jnp.ones((b, n, 1), jnp.float32)
    a = jnp.concatenate([-2.0 * xt, nx, ones_x], axis=2)

    ny = jnp.sum(y * y, axis=1, keepdims=True)
    ones_y = jnp.ones((b, 1, n), jnp.float32)
    bm = jnp.concatenate([y, ones_y, ny], axis=1)

    out_x, out_y = pl.pallas_call(
        _chamfer_kernel,
        grid=(b,),
        in_specs=[
            pl.BlockSpec((None, n, k), lambda i: (i, 0, 0)),
            pl.BlockSpec((None, k, n), lambda i: (i, 0, 0)),
        ],
        out_specs=[
            pl.BlockSpec((None, 1, n), lambda i: (i, 0, 0)),
            pl.BlockSpec((None, 1, n), lambda i: (i, 0, 0)),
        ],
        out_shape=[
            jax.ShapeDtypeStruct((b, 1, n), jnp.float32),
            jax.ShapeDtypeStruct((b, 1, n), jnp.float32),
        ],
    )(a, bm)

    return jnp.mean(out_x) + jnp.mean(out_y)


# speedup vs baseline: 2.0834x; 2.0834x over previous
import jax
import jax.numpy as jnp
from jax.experimental import pallas as pl


_CHUNK = 1024


def _chamfer_kernel(xp_ref, y_ref, out_x_ref, out_y_ref):
    n = y_ref.shape[1]
    d_dims = y_ref.shape[0]
    n_chunks = n // _CHUNK

    y_rows = [y_ref[d : d + 1, :] for d in range(d_dims)]
    ny = y_rows[0] * y_rows[0]
    for d in range(1, d_dims):
        ny = ny + y_rows[d] * y_rows[d]

    ymin = jnp.full((n,), jnp.inf, dtype=jnp.float32)
    for i in range(n_chunks):
        xc = [
            xp_ref[pl.ds(i * _CHUNK, _CHUNK), d : d + 1] for d in range(d_dims)
        ]
        nx = xc[0] * xc[0]
        for d in range(1, d_dims):
            nx = nx + xc[d] * xc[d]
        t = nx + ny
        for d in range(d_dims):
            t = t + (-2.0 * xc[d]) * y_rows[d]
        out_x_ref[0, pl.ds(i * _CHUNK, _CHUNK)] = jnp.min(t, axis=1)
        ymin = jnp.minimum(ymin, jnp.min(t, axis=0))
    out_y_ref[0, :] = ymin


def kernel(x, y):
    b, d, n = x.shape
    xp = jnp.transpose(x, (0, 2, 1))

    out_x, out_y = pl.pallas_call(
        _chamfer_kernel,
        grid=(b,),
        in_specs=[
            pl.BlockSpec((None, n, d), lambda i: (i, 0, 0)),
            pl.BlockSpec((None, d, n), lambda i: (i, 0, 0)),
        ],
        out_specs=[
            pl.BlockSpec((None, 1, n), lambda i: (i, 0, 0)),
            pl.BlockSpec((None, 1, n), lambda i: (i, 0, 0)),
        ],
        out_shape=[
            jax.ShapeDtypeStruct((b, 1, n), jnp.float32),
            jax.ShapeDtypeStruct((b, 1, n), jnp.float32),
        ],
    )(xp, y)

    return jnp.mean(out_x) + jnp.mean(out_y)


# speedup vs baseline: 5.4475x; 2.6148x over previous
import jax
import jax.numpy as jnp
from jax.experimental import pallas as pl


_CHUNK = 512


def _chamfer_kernel(a_ref, bm_ref, out_x_ref, out_y_ref):
    n = bm_ref.shape[1]
    n_chunks = n // _CHUNK

    ymin = jnp.full((n,), jnp.inf, dtype=jnp.float32)
    for i in range(n_chunks):
        a_chunk = a_ref[pl.ds(i * _CHUNK, _CHUNK), :]
        t = jax.lax.dot_general(
            a_chunk,
            bm_ref[...],
            (((1,), (0,)), ((), ())),
            preferred_element_type=jnp.float32,
        )
        out_x_ref[0, pl.ds(i * _CHUNK, _CHUNK)] = jnp.min(t, axis=1)
        ymin = jnp.minimum(ymin, jnp.min(t, axis=0))
    out_y_ref[0, :] = ymin


def kernel(x, y):
    b, d, n = x.shape
    f32 = jnp.float32
    bf16 = jnp.bfloat16

    xt = jnp.transpose(x, (0, 2, 1))
    nx = jnp.sum(xt * xt, axis=2, keepdims=True)
    a_full = jnp.concatenate(
        [-2.0 * xt, nx, jnp.ones((b, n, 1), f32)], axis=2
    )

    ny = jnp.sum(y * y, axis=1, keepdims=True)
    bm_full = jnp.concatenate(
        [y, jnp.ones((b, 1, n), f32), ny], axis=1
    )

    a_hi = a_full.astype(bf16)
    a_lo = (a_full - a_hi.astype(f32)).astype(bf16)
    aa = jnp.concatenate([a_hi, a_hi, a_lo, a_lo], axis=2)

    bm_hi = bm_full.astype(bf16)
    bm_lo = (bm_full - bm_hi.astype(f32)).astype(bf16)
    bb = jnp.concatenate([bm_hi, bm_lo, bm_hi, bm_lo], axis=1)

    k4 = 4 * (d + 2)
    out_x, out_y = pl.pallas_call(
        _chamfer_kernel,
        grid=(b,),
        in_specs=[
            pl.BlockSpec((None, n, k4), lambda i: (i, 0, 0)),
            pl.BlockSpec((None, k4, n), lambda i: (i, 0, 0)),
        ],
        out_specs=[
            pl.BlockSpec((None, 1, n), lambda i: (i, 0, 0)),
            pl.BlockSpec((None, 1, n), lambda i: (i, 0, 0)),
        ],
        out_shape=[
            jax.ShapeDtypeStruct((b, 1, n), f32),
            jax.ShapeDtypeStruct((b, 1, n), f32),
        ],
    )(aa, bb)

    return jnp.mean(out_x) + jnp.mean(out_y)
